# baseline (speedup 1.0000x reference)
import functools
import math

import jax
import jax.numpy as jnp
from jax import lax
from jax.experimental import pallas as pl
from jax.experimental.pallas import tpu as pltpu
from jax.experimental.pallas import tpu_sc as plsc

EMBEDDING_DIM = 64
LANES = 16
NUM_CORES = 2
NUM_SUBCORES = 16
NUM_WORKERS = NUM_CORES * NUM_SUBCORES
SCALE = math.sqrt(EMBEDDING_DIM)
HALF = 524288
BLK = 16384

SEQ = 200
S_SPLIT = 4
S_PER_CHUNK = 2
CHUNK = S_PER_CHUNK * 128


def _prep_body(in1, in2, out):
    ii = lax.broadcasted_iota(jnp.int32, (64, 64), 0)
    jj = lax.broadcasted_iota(jnp.int32, (64, 64), 1)
    ident = jnp.where(ii == jj, jnp.float32(SCALE), jnp.float32(0.0))
    dn = (((0,), (0,)), ((), ()))
    t1 = lax.dot_general(in1[...], ident, dn, preferred_element_type=jnp.float32)
    t2 = lax.dot_general(in2[...], ident, dn, preferred_element_type=jnp.float32)
    out[:, 0:EMBEDDING_DIM] = t1
    out[:, EMBEDDING_DIM : 2 * EMBEDDING_DIM] = t2


@jax.jit
def _tc_pack(wT):
    vocab = wT.shape[1]
    n_in_blocks = (vocab + BLK - 1) // BLK
    return pl.pallas_call(
        _prep_body,
        grid=(HALF // BLK,),
        in_specs=[
            pl.BlockSpec((64, BLK), lambda k: (0, k)),
            pl.BlockSpec(
                (64, BLK),
                lambda k: (0, jnp.minimum(k + HALF // BLK, n_in_blocks - 1)),
            ),
        ],
        out_specs=pl.BlockSpec((BLK, 2 * EMBEDDING_DIM), lambda k: (k, 0)),
        out_shape=jax.ShapeDtypeStruct((HALF, 2 * EMBEDDING_DIM), jnp.float32),
    )(wT, wT)


@jax.jit
def _gather_transposed(table, idx3):
    s_range = SEQ // S_SPLIT
    n_chunks = s_range // S_PER_CHUNK
    mesh = plsc.VectorSubcoreMesh(core_axis_name="c", subcore_axis_name="s")

    @functools.partial(
        pl.kernel,
        mesh=mesh,
        out_type=jax.ShapeDtypeStruct((SEQ, 8, 8, 8, 128), jnp.float32),
        scratch_types=[
            pltpu.VMEM((CHUNK,), jnp.int32),
            pltpu.VMEM((CHUNK,), jnp.int32),
            pltpu.VMEM((CHUNK, EMBEDDING_DIM), jnp.float32),
            pltpu.VMEM((CHUNK, EMBEDDING_DIM), jnp.float32),
            pltpu.VMEM((S_PER_CHUNK * EMBEDDING_DIM, 128), jnp.float32),
            pltpu.VMEM((S_PER_CHUNK * EMBEDDING_DIM, 128), jnp.float32),
            pltpu.SemaphoreType.DMA,
            pltpu.SemaphoreType.DMA,
        ],
        compiler_params=pltpu.CompilerParams(
            use_tc_tiling_on_sc=False, needs_layout_passes=False
        ),
    )
    def gather_kernel(table_hbm, idx_hbm, out_hbm, i0, i1, r0, r1, t0, t1, s0_, s1_):
        wid = lax.axis_index("s") * NUM_CORES + lax.axis_index("c")
        bt = wid // S_SPLIT
        s_base = (wid % S_SPLIT) * s_range
        item_base = bt * (SEQ * 128) + s_base * 128
        idx_v = (i0, i1)
        rows_v = (r0, r1)
        tbuf = (t0, t1)
        sem = (s0_, s1_)

        iota = lax.iota(jnp.int32, LANES)

        def start(g):
            p = g % 2
            off = item_base + g * CHUNK
            pltpu.sync_copy(idx_hbm.at[pl.ds(off, CHUNK)], idx_v[p])
            return pltpu.async_copy(table_hbm.at[idx_v[p]], rows_v[p], sem[p])

        jvec = [jnp.full((LANES,), j, jnp.int32) for j in range(EMBEDDING_DIM)]

        def transpose_chunk(p):
            rows = rows_v[p]
            tb = tbuf[p]

            def body(gr, carry):
                b0 = gr * LANES
                bvec = jnp.full((LANES,), b0, jnp.int32) + iota
                u64 = (b0 >> 7) << 6
                bc0 = b0 & 127
                for j in range(EMBEDDING_DIM):
                    vals = plsc.load_gather(rows, [bvec, jvec[j]])
                    tb[u64 + j, pl.ds(bc0, LANES)] = vals
                return carry

            lax.fori_loop(0, CHUNK // LANES, body, 0)

        pending = start(0)
        for g in range(n_chunks):
            nxt = start(g + 1) if g + 1 < n_chunks else None
            pending.wait()
            p = g % 2
            transpose_chunk(p)
            for u in range(S_PER_CHUNK):
                s = s_base + g * S_PER_CHUNK + u
                for jt in range(8):
                    pltpu.sync_copy(
                        tbuf[p].at[pl.ds(u * EMBEDDING_DIM + jt * 8, 8), :],
                        out_hbm.at[s, jt, bt],
                    )
            pending = nxt

    return gather_kernel(table, idx3)


def kernel(x, weight):
    b, s = x.shape
    total = b * s
    dim = weight.shape[1]
    table2 = _tc_pack(weight.T)
    table = table2.reshape(2 * HALF, dim)
    idxp = x.astype(jnp.int32).reshape(8, 128, s).transpose(0, 2, 1).reshape(total)
    hi = (idxp >= HALF).astype(jnp.int32)
    idx3 = ((idxp - hi * HALF) << 1) + hi
    out5 = _gather_transposed(table, idx3)
    return out5.transpose(2, 4, 0, 1, 3).reshape(b, s, dim)


# speedup vs baseline: 1.5121x; 1.5121x over previous
import functools
import math

import jax
import jax.numpy as jnp
from jax import lax
from jax.experimental import pallas as pl
from jax.experimental.pallas import tpu as pltpu
from jax.experimental.pallas import tpu_sc as plsc

EMBEDDING_DIM = 64
LANES = 16
NUM_CORES = 2
NUM_SUBCORES = 16
NUM_WORKERS = NUM_CORES * NUM_SUBCORES
SCALE = math.sqrt(EMBEDDING_DIM)
HALF = 524288
BLK = 16384


def _prep_body(in1, in2, out):
    ii = lax.broadcasted_iota(jnp.int32, (64, 64), 0)
    jj = lax.broadcasted_iota(jnp.int32, (64, 64), 1)
    ident = jnp.where(ii == jj, jnp.float32(SCALE), jnp.float32(0.0))
    dn = (((0,), (0,)), ((), ()))
    t1 = lax.dot_general(in1[...], ident, dn, preferred_element_type=jnp.float32)
    t2 = lax.dot_general(in2[...], ident, dn, preferred_element_type=jnp.float32)
    out[:, 0:EMBEDDING_DIM] = t1
    out[:, EMBEDDING_DIM : 2 * EMBEDDING_DIM] = t2


@jax.jit
def _tc_pack(wT):
    vocab = wT.shape[1]
    n_in_blocks = (vocab + BLK - 1) // BLK
    return pl.pallas_call(
        _prep_body,
        grid=(HALF // BLK,),
        in_specs=[
            pl.BlockSpec((64, BLK), lambda k: (0, k)),
            pl.BlockSpec(
                (64, BLK),
                lambda k: (0, jnp.minimum(k + HALF // BLK, n_in_blocks - 1)),
            ),
        ],
        out_specs=pl.BlockSpec((BLK, 2 * EMBEDDING_DIM), lambda k: (k, 0)),
        out_shape=jax.ShapeDtypeStruct((HALF, 2 * EMBEDDING_DIM), jnp.float32),
    )(wT, wT)


@functools.partial(jax.jit, static_argnames=("total", "chunk"))
def _gather_rows(table, idx3, *, total, chunk):
    per_worker = total // NUM_WORKERS
    n_chunks = per_worker // chunk
    mesh = plsc.VectorSubcoreMesh(core_axis_name="c", subcore_axis_name="s")

    @functools.partial(
        pl.kernel,
        mesh=mesh,
        out_type=jax.ShapeDtypeStruct((total, EMBEDDING_DIM), jnp.float32),
        scratch_types=[
            pltpu.VMEM((chunk,), jnp.int32),
            pltpu.VMEM((chunk,), jnp.int32),
            pltpu.VMEM((chunk, EMBEDDING_DIM), jnp.float32),
            pltpu.VMEM((chunk, EMBEDDING_DIM), jnp.float32),
            pltpu.SemaphoreType.DMA,
            pltpu.SemaphoreType.DMA,
        ],
        compiler_params=pltpu.CompilerParams(use_tc_tiling_on_sc=False),
    )
    def gather_kernel(table_hbm, idx_hbm, out_hbm, i0, i1, r0, r1, s0, s1):
        wid = lax.axis_index("s") * NUM_CORES + lax.axis_index("c")
        base = wid * per_worker
        idx_v = (i0, i1)
        rows_v = (r0, r1)
        sem = (s0, s1)

        def start(g):
            p = g % 2
            off = base + g * chunk
            pltpu.sync_copy(idx_hbm.at[pl.ds(off, chunk)], idx_v[p])
            return pltpu.async_copy(table_hbm.at[idx_v[p]], rows_v[p], sem[p])

        pending = start(0)
        for g in range(n_chunks):
            nxt = start(g + 1) if g + 1 < n_chunks else None
            pending.wait()
            pltpu.sync_copy(rows_v[g % 2], out_hbm.at[pl.ds(base + g * chunk, chunk)])
            pending = nxt

    return gather_kernel(table, idx3)


def kernel(x, weight):
    b, s = x.shape
    total = b * s
    dim = weight.shape[1]
    idx = x.reshape(total).astype(jnp.int32)
    table2 = _tc_pack(weight.T)
    table = table2.reshape(2 * HALF, EMBEDDING_DIM)
    hi = (idx >= HALF).astype(jnp.int32)
    idx3 = ((idx - hi * HALF) << 1) + hi
    out = _gather_rows(table, idx3, total=total, chunk=800)
    return out.reshape(b, s, dim)
